# baseline (speedup 1.0000x reference)
import functools

import jax
import jax.numpy as jnp
from jax import lax
from jax.experimental import pallas as pl
from jax.experimental.pallas import tpu as pltpu
from jax.experimental.pallas import tpu_sc as plsc

NC = 2
NS = 16
L = 16
NW = NC * NS
B = 16384
D = 64
BPW = B // NW
CHUNK = 128
NCHUNK = BPW // CHUNK
GROUPS = BPW // L


@functools.partial(
    pl.kernel,
    out_type=jax.ShapeDtypeStruct((B,), jnp.float32),
    mesh=plsc.VectorSubcoreMesh(core_axis_name="c", subcore_axis_name="s"),
    compiler_params=pltpu.CompilerParams(needs_layout_passes=False,
                                         use_tc_tiling_on_sc=False),
    scratch_types=[
        pltpu.VMEM((2 * BPW,), jnp.int32),
        pltpu.VMEM((NCHUNK, CHUNK), jnp.int32),
        pltpu.VMEM((NCHUNK, CHUNK), jnp.int32),
        pltpu.VMEM((BPW, D), jnp.float32),
        pltpu.VMEM((BPW, D), jnp.float32),
        pltpu.VMEM((BPW,), jnp.float32),
        pltpu.VMEM((BPW,), jnp.float32),
        pltpu.VMEM((1,), jnp.float32),
        pltpu.VMEM((BPW,), jnp.float32),
        pltpu.VMEM((L, L + 1), jnp.float32),
        pltpu.SemaphoreType.DMA,
    ],
)
def _mf_kernel(x_hbm, p_hbm, q_hbm, bu_hbm, bi_hbm, alfa_hbm, out_hbm,
               x_v, u_idx, i_idx, pu_v, qi_v, bu_v, bi_v, alfa_v, out_v,
               tr_v, sem):
    wid = lax.axis_index("s") * NC + lax.axis_index("c")
    base = wid * BPW
    iota = lax.iota(jnp.int32, L)

    pltpu.sync_copy(x_hbm.at[pl.ds(base * 2, 2 * BPW)], x_v)
    pltpu.sync_copy(alfa_hbm, alfa_v)

    sub = CHUNK // L
    for g in range(GROUPS):
        off = 2 * L * g
        u16 = plsc.load_gather(x_v, [off + 2 * iota])
        i16 = plsc.load_gather(x_v, [off + 2 * iota + 1])
        u_idx[g // sub, pl.ds((g % sub) * L, L)] = u16
        i_idx[g // sub, pl.ds((g % sub) * L, L)] = i16

    copies = []
    for j in range(NCHUNK):
        sl = pl.ds(j * CHUNK, CHUNK)
        copies.append(pltpu.async_copy(p_hbm.at[u_idx.at[j]], pu_v.at[sl], sem))
        copies.append(pltpu.async_copy(q_hbm.at[i_idx.at[j]], qi_v.at[sl], sem))
        copies.append(pltpu.async_copy(bu_hbm.at[u_idx.at[j]], bu_v.at[sl], sem))
        copies.append(pltpu.async_copy(bi_hbm.at[i_idx.at[j]], bi_v.at[sl], sem))
    for c in copies:
        c.wait()

    alfa_s = plsc.load_gather(alfa_v, [iota * 0])

    def group(g, carry):
        b0 = g * L
        for r in range(L):
            row = b0 + r
            acc = pu_v[row, pl.ds(0, L)] * qi_v[row, pl.ds(0, L)]
            for c in range(1, D // L):
                acc = acc + pu_v[row, pl.ds(c * L, L)] * qi_v[row, pl.ds(c * L, L)]
            tr_v[r, pl.ds(0, L)] = acc
        tot = plsc.load_gather(tr_v, [iota, iota * 0])
        for l in range(1, L):
            tot = tot + plsc.load_gather(tr_v, [iota, jnp.full((L,), l, jnp.int32)])
        bu16 = plsc.load_gather(bu_v, [b0 + iota])
        bi16 = plsc.load_gather(bi_v, [b0 + iota])
        out_v[pl.ds(b0, L)] = tot + bu16 + bi16 + alfa_s
        return carry

    lax.fori_loop(0, GROUPS, group, 0)

    pltpu.sync_copy(out_v, out_hbm.at[pl.ds(base, BPW)])


def kernel(x, P, Q, beta_u, beta_i, alfa):
    xf = x.reshape(-1).astype(jnp.int32)
    return _mf_kernel(xf, P, Q, beta_u.reshape(-1), beta_i.reshape(-1),
                      alfa.reshape(-1))
